# baseline (speedup 1.0000x reference)
import functools

import jax
import jax.numpy as jnp
from jax import lax
from jax.experimental import pallas as pl
from jax.experimental.pallas import tpu as pltpu
from jax.experimental.pallas import tpu_sc as plsc

NN = 10000
EE = 320000
DD = 128
HH = 128
OO = 64
GG = 16

N_PAD = 10240
ROWS_PER_TILE = N_PAD // 16
CHUNK = 64
WORKERS = 32
CHUNKS_PER_W = 164
NQUAD = CHUNKS_PER_W // 4
E_CAP = WORKERS * CHUNKS_PER_W * CHUNK

_MESH = plsc.VectorSubcoreMesh(core_axis_name="c", subcore_axis_name="s",
                               num_cores=2, num_subcores=16)


def _zero_acc(zeros_v, acc, sid):
    for k in range(ROWS_PER_TILE // CHUNK):
        row = pl.multiple_of(sid * ROWS_PER_TILE + k * CHUNK, CHUNK)
        pltpu.sync_copy(zeros_v, acc.at[pl.ds(row, CHUNK)])


def _writeback(acc, out, cid, sid):
    row = pl.multiple_of(sid * ROWS_PER_TILE, 8)
    pltpu.sync_copy(acc.at[pl.ds(row, ROWS_PER_TILE)],
                    out.at[cid, pl.ds(row, ROWS_PER_TILE)])


@functools.partial(
    pl.kernel,
    out_type=jax.ShapeDtypeStruct((2, N_PAD, 16), jnp.float32),
    mesh=_MESH,
    scratch_types=[
        pltpu.VMEM((CHUNKS_PER_W // 2, 2 * CHUNK), jnp.int32),
        pltpu.VMEM((CHUNKS_PER_W, CHUNK), jnp.int32),
        pltpu.VMEM((CHUNK, 16), jnp.float32),
        pltpu.VMEM((CHUNK, 16), jnp.float32),
        pltpu.VMEM_SHARED((N_PAD, 16), jnp.float32),
        pltpu.SemaphoreType.DMA,
    ],
)
def _deg_kernel(packed3d, ones_hbm, zeros_hbm, out, packed_v, dsts_v, ones_v,
                zbuf_v, acc, sem):
    cid = lax.axis_index("c")
    sid = lax.axis_index("s")
    wid = sid * 2 + cid
    pltpu.sync_copy(packed3d.at[wid], packed_v)
    pltpu.sync_copy(zeros_hbm, zbuf_v)
    pltpu.sync_copy(ones_hbm, ones_v)
    _zero_acc(zbuf_v, acc, sid)

    def unp(r, carry):
        for h in range(2):
            for j in range(CHUNK // 16):
                p16 = packed_v[r, pl.ds(h * CHUNK + j * 16, 16)]
                dsts_v[2 * r + h, pl.ds(j * 16, 16)] = (
                    lax.shift_right_logical(p16, 14))
        return carry

    lax.fori_loop(0, CHUNKS_PER_W // 2, unp, 0)
    plsc.subcore_barrier()

    def fire(c, carry):
        pltpu.async_copy(ones_v, acc.at[dsts_v.at[c]], sem, add=True)
        return carry

    lax.fori_loop(0, CHUNKS_PER_W, fire, 0)

    def drain(c, carry):
        pltpu.make_async_copy(ones_v, acc.at[dsts_v.at[c]], sem).wait()
        return carry

    lax.fori_loop(0, CHUNKS_PER_W, drain, 0)
    plsc.subcore_barrier()
    _writeback(acc, out, cid, sid)


def _make_agg(w):
    @functools.partial(
        pl.kernel,
        out_type=jax.ShapeDtypeStruct((2, N_PAD, w), jnp.float32),
        mesh=_MESH,
        scratch_types=(
            [pltpu.VMEM((CHUNKS_PER_W // 2, 2 * CHUNK), jnp.int32)]
            + [pltpu.VMEM((CHUNK,), jnp.int32) for _ in range(4)]
            + [pltpu.VMEM((CHUNK,), jnp.int32) for _ in range(4)]
            + [pltpu.VMEM((CHUNK, w), jnp.float32) for _ in range(4)]
            + [pltpu.VMEM_SHARED((N_PAD, w), jnp.float32)]
            + [pltpu.SemaphoreType.DMA for _ in range(8)]
        ),
    )
    def agg(hp, packed3d, zeros_hbm, out, packed_v,
            src0, src1, src2, src3, dst0, dst1, dst2, dst3,
            rows0, rows1, rows2, rows3, acc,
            sg0, sg1, sg2, sg3, ss0, ss1, ss2, ss3):
        cid = lax.axis_index("c")
        sid = lax.axis_index("s")
        wid = sid * 2 + cid
        srcs = [src0, src1, src2, src3]
        dsts = [dst0, dst1, dst2, dst3]
        rows = [rows0, rows1, rows2, rows3]
        sg = [sg0, sg1, sg2, sg3]
        ss = [ss0, ss1, ss2, ss3]

        def unpack(row, half, src_v, dst_v):
            for j in range(CHUNK // 16):
                p16 = packed_v[row, pl.ds(half * CHUNK + j * 16, 16)]
                src_v[pl.ds(j * 16, 16)] = p16 & 0x3FFF
                dst_v[pl.ds(j * 16, 16)] = lax.shift_right_logical(p16, 14)

        pltpu.sync_copy(packed3d.at[wid], packed_v)
        unpack(0, 0, src0, dst0)
        unpack(0, 1, src1, dst1)
        pltpu.async_copy(hp.at[src0], rows0, sg0)
        pltpu.async_copy(hp.at[src1], rows1, sg1)
        pltpu.sync_copy(zeros_hbm, rows3)
        _zero_acc(rows3, acc, sid)
        plsc.subcore_barrier()

        def quad(p, carry):
            for k in range(4):
                c = p * 4 + k
                k2 = (k + 2) % 4
                pltpu.make_async_copy(hp.at[srcs[k]], rows[k], sg[k]).wait()
                pltpu.async_copy(rows[k], acc.at[dsts[k]], ss[k], add=True)

                @pl.when(c >= 2)
                def _():
                    pltpu.make_async_copy(rows[k2], acc.at[dsts[k2]],
                                          ss[k2]).wait()

                @pl.when(c + 2 < CHUNKS_PER_W)
                def _():
                    unpack(p * 2 + (k + 2) // 2, k % 2, srcs[k2], dsts[k2])
                    pltpu.async_copy(hp.at[srcs[k2]], rows[k2], sg[k2])

            return carry

        lax.fori_loop(0, NQUAD, quad, 0)
        pltpu.make_async_copy(rows2, acc.at[dst2], ss2).wait()
        pltpu.make_async_copy(rows3, acc.at[dst3], ss3).wait()
        plsc.subcore_barrier()
        _writeback(acc, out, cid, sid)

    return agg


_agg128 = _make_agg(HH)


_TCGRID = 8
_BLK = N_PAD // _TCGRID


def _tc_h1(x_pad, w1):
    def body(x_ref, w_ref, out_ref):
        out_ref[...] = jnp.dot(x_ref[...], w_ref[...],
                               preferred_element_type=jnp.float32)

    return pl.pallas_call(
        body,
        grid=(_TCGRID,),
        in_specs=[pl.BlockSpec((_BLK, DD), lambda i: (i, 0)),
                  pl.BlockSpec((DD, DD), lambda i: (0, 0))],
        out_specs=pl.BlockSpec((_BLK, DD), lambda i: (i, 0)),
        out_shape=jax.ShapeDtypeStruct((N_PAD, DD), jnp.float32),
    )(x_pad, w1)


def _tc_scale1(deg2, h1):
    def body(deg_ref, h_ref, dinv_ref, hp_ref):
        deg = deg_ref[0, :, 0:1] + deg_ref[1, :, 0:1]
        dinv = jnp.where(deg > 0, lax.rsqrt(deg), 0.0)
        dinv_ref[...] = dinv
        hp_ref[...] = dinv * h_ref[...]

    return pl.pallas_call(
        body,
        grid=(_TCGRID,),
        in_specs=[pl.BlockSpec((2, _BLK, 16), lambda i: (0, i, 0)),
                  pl.BlockSpec((_BLK, DD), lambda i: (i, 0))],
        out_specs=(pl.BlockSpec((_BLK, 1), lambda i: (i, 0)),
                   pl.BlockSpec((_BLK, DD), lambda i: (i, 0))),
        out_shape=(jax.ShapeDtypeStruct((N_PAD, 1), jnp.float32),
                   jax.ShapeDtypeStruct((N_PAD, DD), jnp.float32)),
    )(deg2, h1)


def _tc_mid(acc2, dinv, b_row, w_next):
    def body(acc_ref, dinv_ref, b_ref, w_ref, out_ref):
        dinv = dinv_ref[...]
        a = jnp.maximum(dinv * (acc_ref[0] + acc_ref[1]) + b_ref[...], 0.0)
        out_ref[...] = dinv * jnp.dot(a, w_ref[...],
                                      preferred_element_type=jnp.float32)

    return pl.pallas_call(
        body,
        grid=(_TCGRID,),
        in_specs=[pl.BlockSpec((2, _BLK, HH), lambda i: (0, i, 0)),
                  pl.BlockSpec((_BLK, 1), lambda i: (i, 0)),
                  pl.BlockSpec((1, HH), lambda i: (0, 0)),
                  pl.BlockSpec((HH, HH), lambda i: (0, 0))],
        out_specs=pl.BlockSpec((_BLK, HH), lambda i: (i, 0)),
        out_shape=jax.ShapeDtypeStruct((N_PAD, HH), jnp.float32),
    )(acc2, dinv, b_row, w_next)


def _tc_prep3(acc2, dinv, b_row, x_pad, w3a, w3b):
    def body(acc_ref, dinv_ref, b_ref, x_ref, wa_ref, wb_ref, out_ref):
        dinv = dinv_ref[...]
        a = jnp.maximum(dinv * (acc_ref[0] + acc_ref[1]) + b_ref[...], 0.0)
        hp3 = dinv * (
            jnp.dot(a, wa_ref[...], preferred_element_type=jnp.float32)
            + jnp.dot(x_ref[...], wb_ref[...], preferred_element_type=jnp.float32))
        out_ref[...] = jnp.concatenate(
            [hp3, jnp.zeros((_BLK, HH - OO), jnp.float32)], axis=1)

    return pl.pallas_call(
        body,
        grid=(_TCGRID,),
        in_specs=[pl.BlockSpec((2, _BLK, HH), lambda i: (0, i, 0)),
                  pl.BlockSpec((_BLK, 1), lambda i: (i, 0)),
                  pl.BlockSpec((1, HH), lambda i: (0, 0)),
                  pl.BlockSpec((_BLK, DD), lambda i: (i, 0)),
                  pl.BlockSpec((HH, OO), lambda i: (0, 0)),
                  pl.BlockSpec((DD, OO), lambda i: (0, 0))],
        out_specs=pl.BlockSpec((_BLK, HH), lambda i: (i, 0)),
        out_shape=jax.ShapeDtypeStruct((N_PAD, HH), jnp.float32),
    )(acc2, dinv, b_row, x_pad, w3a, w3b)


def _tc_pool(acc2, dinv, b3_row, gate_w, gate_b11, batch_col):
    def body(acc_ref, dinv_ref, b_ref, gw_ref, gb_ref, batch_ref, out_ref):
        dinv = dinv_ref[...]
        h3 = dinv * (acc_ref[0, :, :OO] + acc_ref[1, :, :OO]) + b_ref[...]
        gate = jnp.dot(h3, gw_ref[...],
                       preferred_element_type=jnp.float32) + gb_ref[0, 0]
        gidx = lax.broadcasted_iota(jnp.int32, (1, GG), 1)
        mask = batch_ref[...] == gidx
        gm = jnp.max(jnp.where(mask, gate, -1e30), axis=0, keepdims=True)
        em = jnp.where(mask, jnp.exp(gate - gm), 0.0)
        denom = jnp.sum(em, axis=0, keepdims=True)
        attn = em / (denom + 1e-16)
        pooled = lax.dot_general(attn, h3, (((0,), (0,)), ((), ())),
                                 preferred_element_type=jnp.float32)
        m = jnp.max(pooled, axis=1, keepdims=True)
        s = jnp.log(jnp.sum(jnp.exp(pooled - m), axis=1, keepdims=True))
        out_ref[...] = pooled - m - s

    return pl.pallas_call(
        body,
        out_shape=jax.ShapeDtypeStruct((GG, OO), jnp.float32),
    )(acc2, dinv, b3_row, gate_w, gate_b11, batch_col)


def kernel(x, edge_index, batch, W1, b1, W2, b2, W3, b3, gate_W, gate_b):
    loop = jnp.arange(NN, dtype=jnp.int32)
    pad_e = E_CAP - (EE + NN)
    pad_idx = NN + (jnp.arange(pad_e, dtype=jnp.int32) % (N_PAD - NN))
    src = jnp.concatenate([edge_index[0], loop, pad_idx])
    dst = jnp.concatenate([edge_index[1], loop, pad_idx])
    packed3d = (src + (dst << 14)).reshape(WORKERS, CHUNKS_PER_W // 2,
                                           2 * CHUNK)
    x_pad = jnp.zeros((N_PAD, DD), jnp.float32).at[:NN].set(x)
    batch_col = jnp.concatenate(
        [batch, jnp.full((N_PAD - NN,), GG, jnp.int32)])[:, None]
    ones16 = jnp.ones((CHUNK, 16), jnp.float32)
    zeros16 = jnp.zeros((CHUNK, 16), jnp.float32)
    zeros128 = jnp.zeros((CHUNK, HH), jnp.float32)

    deg2 = _deg_kernel(packed3d, ones16, zeros16)
    h1 = _tc_h1(x_pad, W1)
    dinv, hp1 = _tc_scale1(deg2, h1)

    acc1 = _agg128(hp1, packed3d, zeros128)
    hp2 = _tc_mid(acc1, dinv, b1[None, :], W2)

    acc2 = _agg128(hp2, packed3d, zeros128)
    hp3 = _tc_prep3(acc2, dinv, b2[None, :], x_pad, W3[:HH], W3[HH:])

    acc3 = _agg128(hp3, packed3d, zeros128)
    return _tc_pool(acc3, dinv, b3[None, :], gate_W, gate_b[None, :], batch_col)


# speedup vs baseline: 1.0163x; 1.0163x over previous
import functools

import jax
import jax.numpy as jnp
from jax import lax
from jax.experimental import pallas as pl
from jax.experimental.pallas import tpu as pltpu
from jax.experimental.pallas import tpu_sc as plsc

NN = 10000
EE = 320000
DD = 128
HH = 128
OO = 64
GG = 16

N_PAD = 10240
ROWS_PER_TILE = N_PAD // 16
CHUNK = 64
WORKERS = 32
CHUNKS_PER_W = 164
NQUAD = CHUNKS_PER_W // 4
E_CAP = WORKERS * CHUNKS_PER_W * CHUNK

_MESH = plsc.VectorSubcoreMesh(core_axis_name="c", subcore_axis_name="s",
                               num_cores=2, num_subcores=16)


def _zero_acc(zeros_v, acc, sid):
    for k in range(ROWS_PER_TILE // CHUNK):
        row = pl.multiple_of(sid * ROWS_PER_TILE + k * CHUNK, CHUNK)
        pltpu.sync_copy(zeros_v, acc.at[pl.ds(row, CHUNK)])


def _writeback(acc, out, cid, sid):
    row = pl.multiple_of(sid * ROWS_PER_TILE, 8)
    pltpu.sync_copy(acc.at[pl.ds(row, ROWS_PER_TILE)],
                    out.at[cid, pl.ds(row, ROWS_PER_TILE)])


@functools.partial(
    pl.kernel,
    out_type=jax.ShapeDtypeStruct((2, N_PAD, 16), jnp.float32),
    mesh=_MESH,
    scratch_types=[
        pltpu.VMEM((CHUNKS_PER_W, CHUNK), jnp.int32),
        pltpu.VMEM((CHUNK, 16), jnp.float32),
        pltpu.VMEM((CHUNK, 16), jnp.float32),
        pltpu.VMEM_SHARED((N_PAD, 16), jnp.float32),
        pltpu.SemaphoreType.DMA,
    ],
)
def _deg_kernel(dst2d, ones_hbm, zeros_hbm, out, dsts_v, ones_v, zbuf_v, acc,
                sem):
    cid = lax.axis_index("c")
    sid = lax.axis_index("s")
    wid = sid * 2 + cid
    pltpu.sync_copy(dst2d.at[wid], dsts_v)
    pltpu.sync_copy(zeros_hbm, zbuf_v)
    pltpu.sync_copy(ones_hbm, ones_v)
    _zero_acc(zbuf_v, acc, sid)
    plsc.subcore_barrier()

    def fire(c, carry):
        pltpu.async_copy(ones_v, acc.at[dsts_v.at[c]], sem, add=True)
        return carry

    lax.fori_loop(0, CHUNKS_PER_W, fire, 0)

    def drain(c, carry):
        pltpu.make_async_copy(ones_v, acc.at[dsts_v.at[c]], sem).wait()
        return carry

    lax.fori_loop(0, CHUNKS_PER_W, drain, 0)
    plsc.subcore_barrier()
    _writeback(acc, out, cid, sid)


def _make_agg(w):
    @functools.partial(
        pl.kernel,
        out_type=jax.ShapeDtypeStruct((2, N_PAD, w), jnp.float32),
        mesh=_MESH,
        scratch_types=(
            [pltpu.VMEM((CHUNKS_PER_W // 2, 2 * CHUNK), jnp.int32)]
            + [pltpu.VMEM((CHUNK,), jnp.int32) for _ in range(4)]
            + [pltpu.VMEM((CHUNK,), jnp.int32) for _ in range(4)]
            + [pltpu.VMEM((CHUNK, w), jnp.float32) for _ in range(4)]
            + [pltpu.VMEM_SHARED((N_PAD, w), jnp.float32)]
            + [pltpu.SemaphoreType.DMA for _ in range(8)]
        ),
    )
    def agg(hp, packed3d, zeros_hbm, out, packed_v,
            src0, src1, src2, src3, dst0, dst1, dst2, dst3,
            rows0, rows1, rows2, rows3, acc,
            sg0, sg1, sg2, sg3, ss0, ss1, ss2, ss3):
        cid = lax.axis_index("c")
        sid = lax.axis_index("s")
        wid = sid * 2 + cid
        srcs = [src0, src1, src2, src3]
        dsts = [dst0, dst1, dst2, dst3]
        rows = [rows0, rows1, rows2, rows3]
        sg = [sg0, sg1, sg2, sg3]
        ss = [ss0, ss1, ss2, ss3]

        def unpack(row, half, src_v, dst_v):
            for j in range(CHUNK // 16):
                p16 = packed_v[row, pl.ds(half * CHUNK + j * 16, 16)]
                src_v[pl.ds(j * 16, 16)] = p16 & 0x3FFF
                dst_v[pl.ds(j * 16, 16)] = lax.shift_right_logical(p16, 14)

        pltpu.sync_copy(packed3d.at[wid], packed_v)
        unpack(0, 0, src0, dst0)
        unpack(0, 1, src1, dst1)
        pltpu.async_copy(hp.at[src0], rows0, sg0)
        pltpu.async_copy(hp.at[src1], rows1, sg1)
        pltpu.sync_copy(zeros_hbm, rows3)
        _zero_acc(rows3, acc, sid)
        plsc.subcore_barrier()

        def quad(p, carry):
            for k in range(4):
                c = p * 4 + k
                k2 = (k + 2) % 4
                pltpu.make_async_copy(hp.at[srcs[k]], rows[k], sg[k]).wait()
                pltpu.async_copy(rows[k], acc.at[dsts[k]], ss[k], add=True)

                @pl.when(c >= 2)
                def _():
                    pltpu.make_async_copy(rows[k2], acc.at[dsts[k2]],
                                          ss[k2]).wait()

                @pl.when(c + 2 < CHUNKS_PER_W)
                def _():
                    unpack(p * 2 + (k + 2) // 2, k % 2, srcs[k2], dsts[k2])
                    pltpu.async_copy(hp.at[srcs[k2]], rows[k2], sg[k2])

            return carry

        lax.fori_loop(0, NQUAD, quad, 0)
        pltpu.make_async_copy(rows2, acc.at[dst2], ss2).wait()
        pltpu.make_async_copy(rows3, acc.at[dst3], ss3).wait()
        plsc.subcore_barrier()
        _writeback(acc, out, cid, sid)

    return agg


_agg128 = _make_agg(HH)


_TCGRID = 8
_BLK = N_PAD // _TCGRID


def _tc_prep1(deg2, x_pad, w1):
    def body(deg_ref, x_ref, w_ref, dinv_ref, hp_ref):
        deg = deg_ref[0, :, 0:1] + deg_ref[1, :, 0:1]
        dinv = jnp.where(deg > 0, lax.rsqrt(deg), 0.0)
        dinv_ref[...] = dinv
        hp_ref[...] = dinv * jnp.dot(x_ref[...], w_ref[...],
                                     preferred_element_type=jnp.float32)

    return pl.pallas_call(
        body,
        grid=(_TCGRID,),
        in_specs=[pl.BlockSpec((2, _BLK, 16), lambda i: (0, i, 0)),
                  pl.BlockSpec((_BLK, DD), lambda i: (i, 0)),
                  pl.BlockSpec((DD, DD), lambda i: (0, 0))],
        out_specs=(pl.BlockSpec((_BLK, 1), lambda i: (i, 0)),
                   pl.BlockSpec((_BLK, DD), lambda i: (i, 0))),
        out_shape=(jax.ShapeDtypeStruct((N_PAD, 1), jnp.float32),
                   jax.ShapeDtypeStruct((N_PAD, DD), jnp.float32)),
    )(deg2, x_pad, w1)


def _tc_mid(acc2, dinv, b_row, w_next):
    def body(acc_ref, dinv_ref, b_ref, w_ref, out_ref):
        dinv = dinv_ref[...]
        a = jnp.maximum(dinv * (acc_ref[0] + acc_ref[1]) + b_ref[...], 0.0)
        out_ref[...] = dinv * jnp.dot(a, w_ref[...],
                                      preferred_element_type=jnp.float32)

    return pl.pallas_call(
        body,
        grid=(_TCGRID,),
        in_specs=[pl.BlockSpec((2, _BLK, HH), lambda i: (0, i, 0)),
                  pl.BlockSpec((_BLK, 1), lambda i: (i, 0)),
                  pl.BlockSpec((1, HH), lambda i: (0, 0)),
                  pl.BlockSpec((HH, HH), lambda i: (0, 0))],
        out_specs=pl.BlockSpec((_BLK, HH), lambda i: (i, 0)),
        out_shape=jax.ShapeDtypeStruct((N_PAD, HH), jnp.float32),
    )(acc2, dinv, b_row, w_next)


def _tc_prep3(acc2, dinv, b_row, x_pad, w3a, w3b):
    def body(acc_ref, dinv_ref, b_ref, x_ref, wa_ref, wb_ref, out_ref):
        dinv = dinv_ref[...]
        a = jnp.maximum(dinv * (acc_ref[0] + acc_ref[1]) + b_ref[...], 0.0)
        hp3 = dinv * (
            jnp.dot(a, wa_ref[...], preferred_element_type=jnp.float32)
            + jnp.dot(x_ref[...], wb_ref[...], preferred_element_type=jnp.float32))
        out_ref[...] = jnp.concatenate(
            [hp3, jnp.zeros((_BLK, HH - OO), jnp.float32)], axis=1)

    return pl.pallas_call(
        body,
        grid=(_TCGRID,),
        in_specs=[pl.BlockSpec((2, _BLK, HH), lambda i: (0, i, 0)),
                  pl.BlockSpec((_BLK, 1), lambda i: (i, 0)),
                  pl.BlockSpec((1, HH), lambda i: (0, 0)),
                  pl.BlockSpec((_BLK, DD), lambda i: (i, 0)),
                  pl.BlockSpec((HH, OO), lambda i: (0, 0)),
                  pl.BlockSpec((DD, OO), lambda i: (0, 0))],
        out_specs=pl.BlockSpec((_BLK, HH), lambda i: (i, 0)),
        out_shape=jax.ShapeDtypeStruct((N_PAD, HH), jnp.float32),
    )(acc2, dinv, b_row, x_pad, w3a, w3b)


def _tc_pool(acc2, dinv, b3_row, gate_w, gate_b11, batch_col):
    def body(acc_ref, dinv_ref, b_ref, gw_ref, gb_ref, batch_ref, out_ref):
        dinv = dinv_ref[...]
        h3 = dinv * (acc_ref[0, :, :OO] + acc_ref[1, :, :OO]) + b_ref[...]
        gate = jnp.dot(h3, gw_ref[...],
                       preferred_element_type=jnp.float32) + gb_ref[0, 0]
        gidx = lax.broadcasted_iota(jnp.int32, (1, GG), 1)
        mask = batch_ref[...] == gidx
        gm = jnp.max(jnp.where(mask, gate, -1e30), axis=0, keepdims=True)
        em = jnp.where(mask, jnp.exp(gate - gm), 0.0)
        denom = jnp.sum(em, axis=0, keepdims=True)
        attn = em / (denom + 1e-16)
        pooled = lax.dot_general(attn, h3, (((0,), (0,)), ((), ())),
                                 preferred_element_type=jnp.float32)
        m = jnp.max(pooled, axis=1, keepdims=True)
        s = jnp.log(jnp.sum(jnp.exp(pooled - m), axis=1, keepdims=True))
        out_ref[...] = pooled - m - s

    return pl.pallas_call(
        body,
        out_shape=jax.ShapeDtypeStruct((GG, OO), jnp.float32),
    )(acc2, dinv, b3_row, gate_w, gate_b11, batch_col)


def kernel(x, edge_index, batch, W1, b1, W2, b2, W3, b3, gate_W, gate_b):
    loop = jnp.arange(NN, dtype=jnp.int32)
    pad_e = E_CAP - (EE + NN)
    pad_idx = NN + (jnp.arange(pad_e, dtype=jnp.int32) % (N_PAD - NN))
    src = jnp.concatenate([edge_index[0], loop, pad_idx])
    dst = jnp.concatenate([edge_index[1], loop, pad_idx])
    packed3d = (src + (dst << 14)).reshape(WORKERS, CHUNKS_PER_W // 2,
                                           2 * CHUNK)
    dst3d = dst.reshape(WORKERS, CHUNKS_PER_W, CHUNK)
    x_pad = jnp.zeros((N_PAD, DD), jnp.float32).at[:NN].set(x)
    batch_col = jnp.concatenate(
        [batch, jnp.full((N_PAD - NN,), GG, jnp.int32)])[:, None]
    ones16 = jnp.ones((CHUNK, 16), jnp.float32)
    zeros16 = jnp.zeros((CHUNK, 16), jnp.float32)
    zeros128 = jnp.zeros((CHUNK, HH), jnp.float32)

    deg2 = _deg_kernel(dst3d, ones16, zeros16)
    dinv, hp1 = _tc_prep1(deg2, x_pad, W1)

    acc1 = _agg128(hp1, packed3d, zeros128)
    hp2 = _tc_mid(acc1, dinv, b1[None, :], W2)

    acc2 = _agg128(hp2, packed3d, zeros128)
    hp3 = _tc_prep3(acc2, dinv, b2[None, :], x_pad, W3[:HH], W3[HH:])

    acc3 = _agg128(hp3, packed3d, zeros128)
    return _tc_pool(acc3, dinv, b3[None, :], gate_W, gate_b[None, :], batch_col)
